# Initial kernel scaffold; baseline (speedup 1.0000x reference)
import jax
import jax.numpy as jnp
from jax.experimental import pallas as pl


def kernel(node, rbf, rsh, edge_index, Wq, Wsrc, Wdst, Wrbf, brbf, Wkv, Wmsg, ln_w, ln_b, Wmlp1, bmlp1, Wmlp2, bmlp2):
    raise NotImplementedError("write your pallas kernel here")


# speedup vs baseline: 6.8637x; 6.8637x over previous
import functools

import jax
import jax.numpy as jnp
import numpy as np
from jax.experimental import pallas as pl
from jax.experimental.pallas import tpu as pltpu

D_NODE = 256
D_EDGE = 16
N_BASIS = 16
D_MSG = 32
N_HEADS = 8
SCALE = 1.0 / np.sqrt(D_MSG)


def _proj_body(node_ref, w_ref, out_ref):
    out_ref[...] = jax.lax.dot(node_ref[...], w_ref[...])


def _edge_body(qs_ref, dg_ref, rbf_ref, rsh_ref, wrbf_ref, brbf_ref,
               wkv_ref, rep32_ref, tile16_ref, shead_ref, rep8_ref, out_ref):
    hi = jax.lax.Precision.HIGHEST
    qg = qs_ref[:, :D_NODE]
    s = qs_ref[:, D_NODE:]
    a = s + dg_ref[...]
    edge = rsh_ref[...] * (
        jax.lax.dot(rbf_ref[...], wrbf_ref[...]) + brbf_ref[...])
    arep = jax.lax.dot(a, rep32_ref[...], precision=hi)
    etile = jax.lax.dot(edge, tile16_ref[...], precision=hi)
    coupled = arep * etile
    kv = jax.lax.dot(coupled, wkv_ref[...])
    kfl = kv[:, :D_NODE]
    vfl = kv[:, D_NODE:]
    proj = jax.lax.dot(qg * kfl, shead_ref[...], precision=hi)
    ex = jnp.exp(proj * SCALE)
    exrep = jax.lax.dot(ex, rep8_ref[...], precision=hi)
    ev = exrep * vfl
    zer = jnp.zeros_like(ex)
    out_ref[...] = jnp.concatenate([ev, ex, zer], axis=1)


def _node_body(nd_ref, acc_ref, wmsg_ref, lnw_ref, lnb_ref,
               w1_ref, b1_ref, w2_ref, b2_ref, rep8_ref, out_ref):
    hi = jax.lax.Precision.HIGHEST
    num = acc_ref[:, :D_NODE]
    den = acc_ref[:, D_NODE:D_NODE + N_HEADS]
    denrep = jax.lax.dot(den, rep8_ref[...], precision=hi)
    msg = num / (denrep + 1e-16)
    message = jax.lax.dot(msg, wmsg_ref[...])
    x0 = jnp.abs(message)
    mu = jnp.mean(x0, axis=-1, keepdims=True)
    var = jnp.mean((x0 - mu) ** 2, axis=-1, keepdims=True)
    x1 = (x0 - mu) * jax.lax.rsqrt(var + 1e-5) * lnw_ref[...] + lnb_ref[...]
    x2 = message / (x0 + 1e-6)
    h = jax.lax.dot(x1, w1_ref[...]) + b1_ref[...]
    h = h / (1.0 + jnp.exp(-h))
    h = jax.lax.dot(h, w2_ref[...]) + b2_ref[...]
    h = h / (1.0 + jnp.exp(-h))
    out_ref[...] = nd_ref[...] + x2 * h


def kernel(node, rbf, rsh, edge_index, Wq, Wsrc, Wdst, Wrbf, brbf, Wkv, Wmsg,
           ln_w, ln_b, Wmlp1, bmlp1, Wmlp2, bmlp2):
    n = node.shape[0]
    E = edge_index.shape[1]
    f32 = jnp.float32
    src_idx = edge_index[0]
    dst_idx = edge_index[1]

    Wcat = jnp.concatenate([Wq, Wsrc, Wdst], axis=1)
    Bn = 1000 if n % 1000 == 0 else n
    proj = pl.pallas_call(
        _proj_body,
        grid=(n // Bn,),
        in_specs=[
            pl.BlockSpec((Bn, D_NODE), lambda i: (i, 0)),
            pl.BlockSpec((D_NODE, 320), lambda i: (0, 0)),
        ],
        out_specs=pl.BlockSpec((Bn, 320), lambda i: (i, 0)),
        out_shape=jax.ShapeDtypeStruct((n, 320), f32),
    )(node, Wcat)
    qs_table = proj[:, :D_NODE + D_MSG]
    d_table = proj[:, D_NODE + D_MSG:]

    qs_g = jnp.take(qs_table, src_idx, axis=0)
    d_g = jnp.take(d_table, dst_idx, axis=0)

    idx5 = jnp.arange(512, dtype=jnp.int32)
    rep32 = (idx5[None, :] // 16 == jnp.arange(32, dtype=jnp.int32)[:, None]).astype(f32)
    tile16 = (idx5[None, :] % 16 == jnp.arange(16, dtype=jnp.int32)[:, None]).astype(f32)
    idx256 = jnp.arange(256, dtype=jnp.int32)
    shead = (idx256[:, None] // 32 == jnp.arange(8, dtype=jnp.int32)[None, :]).astype(f32)
    rep8 = (idx256[None, :] // 32 == jnp.arange(8, dtype=jnp.int32)[:, None]).astype(f32)

    Eb = 800 if E % 800 == 0 else E
    evx = pl.pallas_call(
        _edge_body,
        grid=(E // Eb,),
        in_specs=[
            pl.BlockSpec((Eb, 288), lambda i: (i, 0)),
            pl.BlockSpec((Eb, 32), lambda i: (i, 0)),
            pl.BlockSpec((Eb, N_BASIS), lambda i: (i, 0)),
            pl.BlockSpec((Eb, D_EDGE), lambda i: (i, 0)),
            pl.BlockSpec((N_BASIS, D_EDGE), lambda i: (0, 0)),
            pl.BlockSpec((1, D_EDGE), lambda i: (0, 0)),
            pl.BlockSpec((512, 512), lambda i: (0, 0)),
            pl.BlockSpec((32, 512), lambda i: (0, 0)),
            pl.BlockSpec((16, 512), lambda i: (0, 0)),
            pl.BlockSpec((256, 8), lambda i: (0, 0)),
            pl.BlockSpec((8, 256), lambda i: (0, 0)),
        ],
        out_specs=pl.BlockSpec((Eb, 272), lambda i: (i, 0)),
        out_shape=jax.ShapeDtypeStruct((E, 272), f32),
    )(qs_g, d_g, rbf, rsh, Wrbf, brbf.reshape(1, -1), Wkv,
      rep32, tile16, shead, rep8)

    acc = jax.ops.segment_sum(evx, dst_idx, num_segments=n)

    out = pl.pallas_call(
        _node_body,
        grid=(n // Bn,),
        in_specs=[
            pl.BlockSpec((Bn, D_NODE), lambda i: (i, 0)),
            pl.BlockSpec((Bn, 272), lambda i: (i, 0)),
            pl.BlockSpec((D_NODE, D_NODE), lambda i: (0, 0)),
            pl.BlockSpec((1, D_NODE), lambda i: (0, 0)),
            pl.BlockSpec((1, D_NODE), lambda i: (0, 0)),
            pl.BlockSpec((D_NODE, D_NODE), lambda i: (0, 0)),
            pl.BlockSpec((1, D_NODE), lambda i: (0, 0)),
            pl.BlockSpec((D_NODE, D_NODE), lambda i: (0, 0)),
            pl.BlockSpec((1, D_NODE), lambda i: (0, 0)),
            pl.BlockSpec((8, 256), lambda i: (0, 0)),
        ],
        out_specs=pl.BlockSpec((Bn, D_NODE), lambda i: (i, 0)),
        out_shape=jax.ShapeDtypeStruct((n, D_NODE), f32),
    )(node, acc, Wmsg, ln_w.reshape(1, -1), ln_b.reshape(1, -1),
      Wmlp1, bmlp1.reshape(1, -1), Wmlp2, bmlp2.reshape(1, -1), rep8)
    return out


# speedup vs baseline: 7.9430x; 1.1572x over previous
import functools

import jax
import jax.numpy as jnp
import numpy as np
from jax import lax
from jax.experimental import pallas as pl
from jax.experimental.pallas import tpu as pltpu
from jax.experimental.pallas import tpu_sc as plsc

D_NODE = 256
D_EDGE = 16
N_BASIS = 16
D_MSG = 32
N_HEADS = 8
SCALE = 1.0 / np.sqrt(D_MSG)

_NC, _NS = 2, 16
_NW = _NC * _NS
_CHUNK = 128
_QS = D_NODE + D_MSG
_ROW = 272
_HALF = 5000
_STRIPE = 320


def _proj_body(node_ref, w_ref, out_ref):
    out_ref[...] = jax.lax.dot(node_ref[...], w_ref[...])


def _edge_body(qs_ref, dg_ref, rbf_ref, rsh_ref, wrbf_ref, brbf_ref,
               wkv_ref, rep32_ref, tile16_ref, shead_ref, rep8_ref,
               lo_ref, hi_ref):
    hi = jax.lax.Precision.HIGHEST
    qg = qs_ref[:, :D_NODE]
    s = qs_ref[:, D_NODE:]
    a = s + dg_ref[...]
    edge = rsh_ref[...] * (
        jax.lax.dot(rbf_ref[...], wrbf_ref[...]) + brbf_ref[...])
    arep = jax.lax.dot(a, rep32_ref[...], precision=hi)
    etile = jax.lax.dot(edge, tile16_ref[...], precision=hi)
    coupled = arep * etile
    kv = jax.lax.dot(coupled, wkv_ref[...])
    kfl = kv[:, :D_NODE]
    vfl = kv[:, D_NODE:]
    proj = jax.lax.dot(qg * kfl, shead_ref[...], precision=hi)
    ex = jnp.exp(proj * SCALE)
    exrep = jax.lax.dot(ex, rep8_ref[...], precision=hi)
    ev = exrep * vfl
    zer = jnp.zeros_like(ex)
    lo_ref[...] = ev[:, :128]
    hi_ref[...] = jnp.concatenate([ev[:, 128:], ex, zer], axis=1)


def _node_body(nd_ref, lo_ref, hi_ref, wmsg_ref, lnw_ref, lnb_ref,
               w1_ref, b1_ref, w2_ref, b2_ref, rep8_ref, out_ref):
    hi = jax.lax.Precision.HIGHEST
    num = jnp.concatenate([lo_ref[...], hi_ref[:, :128]], axis=1)
    den = hi_ref[:, 128:128 + N_HEADS]
    denrep = jax.lax.dot(den, rep8_ref[...], precision=hi)
    msg = num / (denrep + 1e-16)
    message = jax.lax.dot(msg, wmsg_ref[...])
    x0 = jnp.abs(message)
    mu = jnp.mean(x0, axis=-1, keepdims=True)
    var = jnp.mean((x0 - mu) ** 2, axis=-1, keepdims=True)
    x1 = (x0 - mu) * jax.lax.rsqrt(var + 1e-5) * lnw_ref[...] + lnb_ref[...]
    x2 = message / (x0 + 1e-6)
    h = jax.lax.dot(x1, w1_ref[...]) + b1_ref[...]
    h = h / (1.0 + jnp.exp(-h))
    h = jax.lax.dot(h, w2_ref[...]) + b2_ref[...]
    h = h / (1.0 + jnp.exp(-h))
    out_ref[...] = nd_ref[...] + x2 * h


def _gather_body(qs_hbm, d_hbm, src_hbm, dstg_hbm, qs_out, d_out,
                 idx_s, idx_d, qs_rows, d_rows, sem1, sem2):
    gj = src_hbm.shape[1]
    c = lax.axis_index("c")
    s = lax.axis_index("s")
    wid = c * _NS + s
    base = wid * (gj * _CHUNK)
    pltpu.sync_copy(src_hbm.at[wid], idx_s)
    pltpu.sync_copy(dstg_hbm.at[wid], idx_d)

    def body(j, carry):
        pltpu.async_copy(qs_hbm.at[idx_s.at[j]], qs_rows, sem1).wait()
        pltpu.async_copy(d_hbm.at[idx_d.at[j]], d_rows, sem2).wait()
        pltpu.sync_copy(qs_rows, qs_out.at[pl.ds(base + j * _CHUNK, _CHUNK)])
        pltpu.sync_copy(d_rows, d_out.at[pl.ds(base + j * _CHUNK, _CHUNK)])
        return carry

    lax.fori_loop(0, gj, body, 0)


def _scatter_body(evx_hbm, dsts_hbm, zeros_hbm, out_hbm,
                  idx_v, idx2_v, ev_v, acc, sem):
    sj = dsts_hbm.shape[1]
    c = lax.axis_index("c")
    s = lax.axis_index("s")
    half = c * _HALF
    pltpu.sync_copy(zeros_hbm, acc.at[pl.ds(s * _STRIPE, _STRIPE)])
    pltpu.sync_copy(dsts_hbm.at[s], idx_v)

    def tbody(j, carry):
        for k in range(_CHUNK // 16):
            v = idx_v[j, pl.ds(k * 16, 16)]
            loc = v - half
            ok = (loc >= 0) & (loc < _HALF)
            idx2_v[j, pl.ds(k * 16, 16)] = jnp.where(ok, loc, _HALF)
        return carry

    lax.fori_loop(0, sj, tbody, 0)
    plsc.subcore_barrier()

    base = s * (sj * _CHUNK)

    def body(j, carry):
        pltpu.sync_copy(evx_hbm.at[pl.ds(base + j * _CHUNK, _CHUNK)], ev_v)
        pltpu.sync_copy(ev_v, acc.at[idx2_v.at[j]], add=True)
        return carry

    lax.fori_loop(0, sj, body, 0)
    plsc.subcore_barrier()

    @pl.when(s < _NS - 1)
    def _():
        pltpu.sync_copy(acc.at[pl.ds(s * _STRIPE, _STRIPE)],
                        out_hbm.at[pl.ds(half + s * _STRIPE, _STRIPE)])

    @pl.when(s == _NS - 1)
    def _():
        last = _HALF - (_NS - 1) * _STRIPE
        pltpu.sync_copy(acc.at[pl.ds((_NS - 1) * _STRIPE, last)],
                        out_hbm.at[pl.ds(half + (_NS - 1) * _STRIPE, last)])


def kernel(node, rbf, rsh, edge_index, Wq, Wsrc, Wdst, Wrbf, brbf, Wkv, Wmsg,
           ln_w, ln_b, Wmlp1, bmlp1, Wmlp2, bmlp2):
    n = node.shape[0]
    E = edge_index.shape[1]
    f32 = jnp.float32
    src_idx = edge_index[0]
    dst_idx = edge_index[1]

    epad = ((E + _NW * _CHUNK - 1) // (_NW * _CHUNK)) * (_NW * _CHUNK)
    gj = epad // _NW // _CHUNK
    sj = epad // _NS // _CHUNK
    src3 = jnp.pad(src_idx, (0, epad - E)).reshape(_NW, gj, _CHUNK)
    dstg3 = jnp.pad(dst_idx, (0, epad - E)).reshape(_NW, gj, _CHUNK)
    dsts3 = jnp.pad(dst_idx, (0, epad - E),
                    constant_values=n).reshape(_NS, sj, _CHUNK)
    rbf_p = jnp.pad(rbf, ((0, epad - E), (0, 0)))
    rsh_p = jnp.pad(rsh, ((0, epad - E), (0, 0)))

    Wcat = jnp.concatenate([Wq, Wsrc, Wdst], axis=1)
    Bn = 1000 if n % 1000 == 0 else n
    proj = pl.pallas_call(
        _proj_body,
        grid=(n // Bn,),
        in_specs=[
            pl.BlockSpec((Bn, D_NODE), lambda i: (i, 0)),
            pl.BlockSpec((D_NODE, 320), lambda i: (0, 0)),
        ],
        out_specs=pl.BlockSpec((Bn, 320), lambda i: (i, 0)),
        out_shape=jax.ShapeDtypeStruct((n, 320), f32),
    )(node, Wcat)
    qs_table = proj[:, :_QS]
    d_table = proj[:, _QS:]

    gather = pl.kernel(
        _gather_body,
        out_type=[jax.ShapeDtypeStruct((epad, _QS), f32),
                  jax.ShapeDtypeStruct((epad, D_MSG), f32)],
        mesh=plsc.VectorSubcoreMesh(core_axis_name="c", subcore_axis_name="s"),
        compiler_params=pltpu.CompilerParams(use_tc_tiling_on_sc=False),
        scratch_types=[
            pltpu.VMEM((gj, _CHUNK), jnp.int32),
            pltpu.VMEM((gj, _CHUNK), jnp.int32),
            pltpu.VMEM((_CHUNK, _QS), f32),
            pltpu.VMEM((_CHUNK, D_MSG), f32),
            pltpu.SemaphoreType.DMA,
            pltpu.SemaphoreType.DMA,
        ],
    )
    qs_g, d_g = gather(qs_table, d_table, src3, dstg3)

    idx5 = jnp.arange(512, dtype=jnp.int32)
    rep32 = (idx5[None, :] // 16 == jnp.arange(32, dtype=jnp.int32)[:, None]).astype(f32)
    tile16 = (idx5[None, :] % 16 == jnp.arange(16, dtype=jnp.int32)[:, None]).astype(f32)
    idx256 = jnp.arange(256, dtype=jnp.int32)
    shead = (idx256[:, None] // 32 == jnp.arange(8, dtype=jnp.int32)[None, :]).astype(f32)
    rep8 = (idx256[None, :] // 32 == jnp.arange(8, dtype=jnp.int32)[:, None]).astype(f32)

    Eb = 1024
    evx = pl.pallas_call(
        _edge_body,
        grid=(epad // Eb,),
        in_specs=[
            pl.BlockSpec((Eb, _QS), lambda i: (i, 0)),
            pl.BlockSpec((Eb, D_MSG), lambda i: (i, 0)),
            pl.BlockSpec((Eb, N_BASIS), lambda i: (i, 0)),
            pl.BlockSpec((Eb, D_EDGE), lambda i: (i, 0)),
            pl.BlockSpec((N_BASIS, D_EDGE), lambda i: (0, 0)),
            pl.BlockSpec((1, D_EDGE), lambda i: (0, 0)),
            pl.BlockSpec((512, 512), lambda i: (0, 0)),
            pl.BlockSpec((32, 512), lambda i: (0, 0)),
            pl.BlockSpec((16, 512), lambda i: (0, 0)),
            pl.BlockSpec((256, 8), lambda i: (0, 0)),
            pl.BlockSpec((8, 256), lambda i: (0, 0)),
        ],
        out_specs=[pl.BlockSpec((Eb, 128), lambda i: (i, 0)),
                   pl.BlockSpec((Eb, 144), lambda i: (i, 0))],
        out_shape=[jax.ShapeDtypeStruct((epad, 128), f32),
                   jax.ShapeDtypeStruct((epad, 144), f32)],
    )(qs_g, d_g, rbf_p, rsh_p, Wrbf, brbf.reshape(1, -1), Wkv,
      rep32, tile16, shead, rep8)
    evx_lo, evx_hi = evx

    def make_scatter(w):
        return pl.kernel(
            _scatter_body,
            out_type=jax.ShapeDtypeStruct((n, w), f32),
            mesh=plsc.VectorSubcoreMesh(core_axis_name="c",
                                        subcore_axis_name="s"),
            compiler_params=pltpu.CompilerParams(use_tc_tiling_on_sc=False),
            scratch_types=[
                pltpu.VMEM((sj, _CHUNK), jnp.int32),
                pltpu.VMEM((sj, _CHUNK), jnp.int32),
                pltpu.VMEM((_CHUNK, w), f32),
                pltpu.VMEM_SHARED((_NS * _STRIPE, w), f32),
                pltpu.SemaphoreType.DMA,
            ],
        )

    acc_lo = make_scatter(128)(evx_lo, dsts3, jnp.zeros((_STRIPE, 128), f32))
    acc_hi = make_scatter(144)(evx_hi, dsts3, jnp.zeros((_STRIPE, 144), f32))

    out = pl.pallas_call(
        _node_body,
        grid=(n // Bn,),
        in_specs=[
            pl.BlockSpec((Bn, D_NODE), lambda i: (i, 0)),
            pl.BlockSpec((Bn, 128), lambda i: (i, 0)),
            pl.BlockSpec((Bn, 144), lambda i: (i, 0)),
            pl.BlockSpec((D_NODE, D_NODE), lambda i: (0, 0)),
            pl.BlockSpec((1, D_NODE), lambda i: (0, 0)),
            pl.BlockSpec((1, D_NODE), lambda i: (0, 0)),
            pl.BlockSpec((D_NODE, D_NODE), lambda i: (0, 0)),
            pl.BlockSpec((1, D_NODE), lambda i: (0, 0)),
            pl.BlockSpec((D_NODE, D_NODE), lambda i: (0, 0)),
            pl.BlockSpec((1, D_NODE), lambda i: (0, 0)),
            pl.BlockSpec((8, 256), lambda i: (0, 0)),
        ],
        out_specs=pl.BlockSpec((Bn, D_NODE), lambda i: (i, 0)),
        out_shape=jax.ShapeDtypeStruct((n, D_NODE), f32),
    )(node, acc_lo, acc_hi, Wmsg, ln_w.reshape(1, -1), ln_b.reshape(1, -1),
      Wmlp1, bmlp1.reshape(1, -1), Wmlp2, bmlp2.reshape(1, -1), rep8)
    return out


# speedup vs baseline: 8.3725x; 1.0541x over previous
import functools

import jax
import jax.numpy as jnp
import numpy as np
from jax import lax
from jax.experimental import pallas as pl
from jax.experimental.pallas import tpu as pltpu
from jax.experimental.pallas import tpu_sc as plsc

D_NODE = 256
D_EDGE = 16
N_BASIS = 16
D_MSG = 32
N_HEADS = 8
SCALE = 1.0 / np.sqrt(D_MSG)

_NC, _NS = 2, 16
_NW = _NC * _NS
_CHUNK = 128
_QS = D_NODE + D_MSG
_ROW = 272
_HALF = 5000
_STRIPE = 320


def _proj_body(node_ref, w_ref, out_ref):
    out_ref[...] = jax.lax.dot(node_ref[...], w_ref[...])


def _edge_body(qs_ref, dg_ref, rbf_ref, rsh_ref, wrbf_ref, brbf_ref,
               wkv_ref, rep32_ref, tile16_ref, shead_ref, rep8_ref,
               lo_ref, hi_ref):
    hi = jax.lax.Precision.HIGHEST
    qg = qs_ref[:, :D_NODE]
    s = qs_ref[:, D_NODE:]
    a = s + dg_ref[...]
    edge = rsh_ref[...] * (
        jax.lax.dot(rbf_ref[...], wrbf_ref[...]) + brbf_ref[...])
    arep = jax.lax.dot(a, rep32_ref[...], precision=hi)
    etile = jax.lax.dot(edge, tile16_ref[...], precision=hi)
    coupled = arep * etile
    kv = jax.lax.dot(coupled, wkv_ref[...])
    kfl = kv[:, :D_NODE]
    vfl = kv[:, D_NODE:]
    proj = jax.lax.dot(qg * kfl, shead_ref[...], precision=hi)
    ex = jnp.exp(proj * SCALE)
    exrep = jax.lax.dot(ex, rep8_ref[...], precision=hi)
    ev = exrep * vfl
    zer = jnp.zeros_like(ex)
    lo_ref[...] = ev[:, :128]
    hi_ref[...] = jnp.concatenate([ev[:, 128:], ex, zer], axis=1)


def _node_body(nd_ref, lo_ref, hi_ref, wmsg_ref, lnw_ref, lnb_ref,
               w1_ref, b1_ref, w2_ref, b2_ref, rep8_ref, out_ref):
    hi = jax.lax.Precision.HIGHEST
    num = jnp.concatenate([lo_ref[...], hi_ref[:, :128]], axis=1)
    den = hi_ref[:, 128:128 + N_HEADS]
    denrep = jax.lax.dot(den, rep8_ref[...], precision=hi)
    msg = num / (denrep + 1e-16)
    message = jax.lax.dot(msg, wmsg_ref[...])
    x0 = jnp.abs(message)
    mu = jnp.mean(x0, axis=-1, keepdims=True)
    var = jnp.mean((x0 - mu) ** 2, axis=-1, keepdims=True)
    x1 = (x0 - mu) * jax.lax.rsqrt(var + 1e-5) * lnw_ref[...] + lnb_ref[...]
    x2 = message / (x0 + 1e-6)
    h = jax.lax.dot(x1, w1_ref[...]) + b1_ref[...]
    h = h / (1.0 + jnp.exp(-h))
    h = jax.lax.dot(h, w2_ref[...]) + b2_ref[...]
    h = h / (1.0 + jnp.exp(-h))
    out_ref[...] = nd_ref[...] + x2 * h


def _gather_body(qs_hbm, d_hbm, src_hbm, dstg_hbm, qs_out, d_out,
                 idx_s, idx_d, qs_rows, d_rows, gsem, csem):
    gj = src_hbm.shape[1]
    c = lax.axis_index("c")
    s = lax.axis_index("s")
    wid = c * _NS + s
    base = wid * (gj * _CHUNK)
    pltpu.sync_copy(src_hbm.at[wid], idx_s)
    pltpu.sync_copy(dstg_hbm.at[wid], idx_d)

    def fire(j):
        sl = j % 2
        pltpu.async_copy(qs_hbm.at[idx_s.at[j]],
                         qs_rows.at[pl.ds(sl * _CHUNK, _CHUNK)], gsem.at[sl])
        pltpu.async_copy(d_hbm.at[idx_d.at[j]],
                         d_rows.at[pl.ds(sl * _CHUNK, _CHUNK)], gsem.at[sl])

    def drain_gather(j):
        sl = j % 2
        pltpu.make_async_copy(qs_hbm.at[idx_s.at[j]],
                              qs_rows.at[pl.ds(sl * _CHUNK, _CHUNK)],
                              gsem.at[sl]).wait()
        pltpu.make_async_copy(d_hbm.at[idx_d.at[j]],
                              d_rows.at[pl.ds(sl * _CHUNK, _CHUNK)],
                              gsem.at[sl]).wait()

    def fire_out(j):
        sl = j % 2
        row = base + j * _CHUNK
        pltpu.async_copy(qs_rows.at[pl.ds(sl * _CHUNK, _CHUNK)],
                         qs_out.at[pl.ds(row, _CHUNK)], csem.at[sl])
        pltpu.async_copy(d_rows.at[pl.ds(sl * _CHUNK, _CHUNK)],
                         d_out.at[pl.ds(row, _CHUNK)], csem.at[sl])

    def drain_out(j):
        sl = j % 2
        row = base + j * _CHUNK
        pltpu.make_async_copy(qs_rows.at[pl.ds(sl * _CHUNK, _CHUNK)],
                              qs_out.at[pl.ds(row, _CHUNK)], csem.at[sl]).wait()
        pltpu.make_async_copy(d_rows.at[pl.ds(sl * _CHUNK, _CHUNK)],
                              d_out.at[pl.ds(row, _CHUNK)], csem.at[sl]).wait()

    fire(0)
    for j in range(gj):
        if j + 1 < gj:
            if j + 1 >= 2:
                drain_out(j - 1)
            fire(j + 1)
        drain_gather(j)
        fire_out(j)
    drain_out(gj - 1)


def _scatter_body(evx_hbm, dsts_hbm, zeros_hbm, out_hbm,
                  idx_v, idx2_v, ev_v, acc, sem):
    sj = dsts_hbm.shape[1]
    c = lax.axis_index("c")
    s = lax.axis_index("s")
    half = c * _HALF
    pltpu.sync_copy(zeros_hbm, acc.at[pl.ds(s * _STRIPE, _STRIPE)])
    pltpu.sync_copy(dsts_hbm.at[s], idx_v)

    def tbody(j, carry):
        for k in range(_CHUNK // 16):
            v = idx_v[j, pl.ds(k * 16, 16)]
            loc = v - half
            ok = (loc >= 0) & (loc < _HALF)
            idx2_v[j, pl.ds(k * 16, 16)] = jnp.where(ok, loc, _HALF)
        return carry

    lax.fori_loop(0, sj, tbody, 0)
    plsc.subcore_barrier()

    base = s * (sj * _CHUNK)

    def body(j, carry):
        pltpu.sync_copy(evx_hbm.at[pl.ds(base + j * _CHUNK, _CHUNK)], ev_v)
        pltpu.sync_copy(ev_v, acc.at[idx2_v.at[j]], add=True)
        return carry

    lax.fori_loop(0, sj, body, 0)
    plsc.subcore_barrier()

    @pl.when(s < _NS - 1)
    def _():
        pltpu.sync_copy(acc.at[pl.ds(s * _STRIPE, _STRIPE)],
                        out_hbm.at[pl.ds(half + s * _STRIPE, _STRIPE)])

    @pl.when(s == _NS - 1)
    def _():
        last = _HALF - (_NS - 1) * _STRIPE
        pltpu.sync_copy(acc.at[pl.ds((_NS - 1) * _STRIPE, last)],
                        out_hbm.at[pl.ds(half + (_NS - 1) * _STRIPE, last)])


def kernel(node, rbf, rsh, edge_index, Wq, Wsrc, Wdst, Wrbf, brbf, Wkv, Wmsg,
           ln_w, ln_b, Wmlp1, bmlp1, Wmlp2, bmlp2):
    n = node.shape[0]
    E = edge_index.shape[1]
    f32 = jnp.float32
    src_idx = edge_index[0]
    dst_idx = edge_index[1]

    epad = ((E + _NW * _CHUNK - 1) // (_NW * _CHUNK)) * (_NW * _CHUNK)
    gj = epad // _NW // _CHUNK
    sj = epad // _NS // _CHUNK
    src3 = jnp.pad(src_idx, (0, epad - E)).reshape(_NW, gj, _CHUNK)
    dstg3 = jnp.pad(dst_idx, (0, epad - E)).reshape(_NW, gj, _CHUNK)
    dsts3 = jnp.pad(dst_idx, (0, epad - E),
                    constant_values=n).reshape(_NS, sj, _CHUNK)
    rbf_p = jnp.pad(rbf, ((0, epad - E), (0, 0)))
    rsh_p = jnp.pad(rsh, ((0, epad - E), (0, 0)))

    Wcat = jnp.concatenate([Wq, Wsrc, Wdst], axis=1)
    Bn = 1000 if n % 1000 == 0 else n
    proj = pl.pallas_call(
        _proj_body,
        grid=(n // Bn,),
        in_specs=[
            pl.BlockSpec((Bn, D_NODE), lambda i: (i, 0)),
            pl.BlockSpec((D_NODE, 320), lambda i: (0, 0)),
        ],
        out_specs=pl.BlockSpec((Bn, 320), lambda i: (i, 0)),
        out_shape=jax.ShapeDtypeStruct((n, 320), f32),
    )(node, Wcat)
    qs_table = proj[:, :_QS]
    d_table = proj[:, _QS:]

    gather = pl.kernel(
        _gather_body,
        out_type=[jax.ShapeDtypeStruct((epad, _QS), f32),
                  jax.ShapeDtypeStruct((epad, D_MSG), f32)],
        mesh=plsc.VectorSubcoreMesh(core_axis_name="c", subcore_axis_name="s"),
        compiler_params=pltpu.CompilerParams(use_tc_tiling_on_sc=False),
        scratch_types=[
            pltpu.VMEM((gj, _CHUNK), jnp.int32),
            pltpu.VMEM((gj, _CHUNK), jnp.int32),
            pltpu.VMEM((2 * _CHUNK, _QS), f32),
            pltpu.VMEM((2 * _CHUNK, D_MSG), f32),
            pltpu.SemaphoreType.DMA((2,)),
            pltpu.SemaphoreType.DMA((2,)),
        ],
    )
    qs_g, d_g = gather(qs_table, d_table, src3, dstg3)

    idx5 = jnp.arange(512, dtype=jnp.int32)
    rep32 = (idx5[None, :] // 16 == jnp.arange(32, dtype=jnp.int32)[:, None]).astype(f32)
    tile16 = (idx5[None, :] % 16 == jnp.arange(16, dtype=jnp.int32)[:, None]).astype(f32)
    idx256 = jnp.arange(256, dtype=jnp.int32)
    shead = (idx256[:, None] // 32 == jnp.arange(8, dtype=jnp.int32)[None, :]).astype(f32)
    rep8 = (idx256[None, :] // 32 == jnp.arange(8, dtype=jnp.int32)[:, None]).astype(f32)

    Eb = 1024
    evx = pl.pallas_call(
        _edge_body,
        grid=(epad // Eb,),
        in_specs=[
            pl.BlockSpec((Eb, _QS), lambda i: (i, 0)),
            pl.BlockSpec((Eb, D_MSG), lambda i: (i, 0)),
            pl.BlockSpec((Eb, N_BASIS), lambda i: (i, 0)),
            pl.BlockSpec((Eb, D_EDGE), lambda i: (i, 0)),
            pl.BlockSpec((N_BASIS, D_EDGE), lambda i: (0, 0)),
            pl.BlockSpec((1, D_EDGE), lambda i: (0, 0)),
            pl.BlockSpec((512, 512), lambda i: (0, 0)),
            pl.BlockSpec((32, 512), lambda i: (0, 0)),
            pl.BlockSpec((16, 512), lambda i: (0, 0)),
            pl.BlockSpec((256, 8), lambda i: (0, 0)),
            pl.BlockSpec((8, 256), lambda i: (0, 0)),
        ],
        out_specs=[pl.BlockSpec((Eb, 128), lambda i: (i, 0)),
                   pl.BlockSpec((Eb, 144), lambda i: (i, 0))],
        out_shape=[jax.ShapeDtypeStruct((epad, 128), f32),
                   jax.ShapeDtypeStruct((epad, 144), f32)],
    )(qs_g, d_g, rbf_p, rsh_p, Wrbf, brbf.reshape(1, -1), Wkv,
      rep32, tile16, shead, rep8)
    evx_lo, evx_hi = evx

    def make_scatter(w):
        return pl.kernel(
            _scatter_body,
            out_type=jax.ShapeDtypeStruct((n, w), f32),
            mesh=plsc.VectorSubcoreMesh(core_axis_name="c",
                                        subcore_axis_name="s"),
            compiler_params=pltpu.CompilerParams(use_tc_tiling_on_sc=False),
            scratch_types=[
                pltpu.VMEM((sj, _CHUNK), jnp.int32),
                pltpu.VMEM((sj, _CHUNK), jnp.int32),
                pltpu.VMEM((_CHUNK, w), f32),
                pltpu.VMEM_SHARED((_NS * _STRIPE, w), f32),
                pltpu.SemaphoreType.DMA,
            ],
        )

    acc_lo = make_scatter(128)(evx_lo, dsts3, jnp.zeros((_STRIPE, 128), f32))
    acc_hi = make_scatter(144)(evx_hi, dsts3, jnp.zeros((_STRIPE, 144), f32))

    out = pl.pallas_call(
        _node_body,
        grid=(n // Bn,),
        in_specs=[
            pl.BlockSpec((Bn, D_NODE), lambda i: (i, 0)),
            pl.BlockSpec((Bn, 128), lambda i: (i, 0)),
            pl.BlockSpec((Bn, 144), lambda i: (i, 0)),
            pl.BlockSpec((D_NODE, D_NODE), lambda i: (0, 0)),
            pl.BlockSpec((1, D_NODE), lambda i: (0, 0)),
            pl.BlockSpec((1, D_NODE), lambda i: (0, 0)),
            pl.BlockSpec((D_NODE, D_NODE), lambda i: (0, 0)),
            pl.BlockSpec((1, D_NODE), lambda i: (0, 0)),
            pl.BlockSpec((D_NODE, D_NODE), lambda i: (0, 0)),
            pl.BlockSpec((1, D_NODE), lambda i: (0, 0)),
            pl.BlockSpec((8, 256), lambda i: (0, 0)),
        ],
        out_specs=pl.BlockSpec((Bn, D_NODE), lambda i: (i, 0)),
        out_shape=jax.ShapeDtypeStruct((n, D_NODE), f32),
    )(node, acc_lo, acc_hi, Wmsg, ln_w.reshape(1, -1), ln_b.reshape(1, -1),
      Wmlp1, bmlp1.reshape(1, -1), Wmlp2, bmlp2.reshape(1, -1), rep8)
    return out


# speedup vs baseline: 9.3076x; 1.1117x over previous
import functools

import jax
import jax.numpy as jnp
import numpy as np
from jax import lax
from jax.experimental import pallas as pl
from jax.experimental.pallas import tpu as pltpu
from jax.experimental.pallas import tpu_sc as plsc

D_NODE = 256
D_EDGE = 16
N_BASIS = 16
D_MSG = 32
N_HEADS = 8
SCALE = 1.0 / np.sqrt(D_MSG)

_NC, _NS = 2, 16
_NW = _NC * _NS
_CHUNK = 128
_QS = D_NODE + D_MSG
_ROW = 272
_HALF = 5000
_STRIPE = 320


def _proj_body(node_ref, w_ref, out_ref):
    out_ref[...] = jax.lax.dot(node_ref[...], w_ref[...])


def _edge_body(qs_ref, dg_ref, rbf_ref, rsh_ref, wrbf_ref, brbf_ref,
               wkv_ref, rep32_ref, tile16_ref, shead_ref, rep8_ref,
               lo_ref, hi_ref):
    hi = jax.lax.Precision.HIGHEST
    qg = qs_ref[:, :D_NODE]
    s = qs_ref[:, D_NODE:]
    a = s + dg_ref[...]
    edge = rsh_ref[...] * (
        jax.lax.dot(rbf_ref[...], wrbf_ref[...]) + brbf_ref[...])
    f32 = jnp.float32
    bf16 = jnp.bfloat16

    def _rep_exact(x, sel_bf16):
        xh = x.astype(bf16)
        r1 = x - xh.astype(f32)
        xm = r1.astype(bf16)
        xl = (r1 - xm.astype(f32)).astype(bf16)
        out = jax.lax.dot(xh, sel_bf16, preferred_element_type=f32)
        out = out + jax.lax.dot(xm, sel_bf16, preferred_element_type=f32)
        return out + jax.lax.dot(xl, sel_bf16, preferred_element_type=f32)

    arep = _rep_exact(a, rep32_ref[...])
    etile = _rep_exact(edge, tile16_ref[...])
    coupled = arep * etile
    kv = jax.lax.dot(coupled, wkv_ref[...])
    kfl = kv[:, :D_NODE]
    vfl = kv[:, D_NODE:]
    proj = jax.lax.dot(qg * kfl, shead_ref[...], precision=hi)
    ex = jnp.exp(proj * SCALE)
    exrep = jax.lax.dot(ex, rep8_ref[...], precision=hi)
    ev = exrep * vfl
    zer = jnp.zeros_like(ex)
    lo_ref[...] = ev[:, :128]
    hi_ref[...] = jnp.concatenate([ev[:, 128:], ex, zer], axis=1)


def _node_body(nd_ref, lo_ref, hi_ref, wmsg_ref, lnw_ref, lnb_ref,
               w1_ref, b1_ref, w2_ref, b2_ref, rep8_ref, out_ref):
    hi = jax.lax.Precision.HIGHEST
    num = jnp.concatenate([lo_ref[...], hi_ref[:, :128]], axis=1)
    den = hi_ref[:, 128:128 + N_HEADS]
    denrep = jax.lax.dot(den, rep8_ref[...], precision=hi)
    msg = num / (denrep + 1e-16)
    message = jax.lax.dot(msg, wmsg_ref[...])
    x0 = jnp.abs(message)
    mu = jnp.mean(x0, axis=-1, keepdims=True)
    var = jnp.mean((x0 - mu) ** 2, axis=-1, keepdims=True)
    x1 = (x0 - mu) * jax.lax.rsqrt(var + 1e-5) * lnw_ref[...] + lnb_ref[...]
    x2 = message / (x0 + 1e-6)
    h = jax.lax.dot(x1, w1_ref[...]) + b1_ref[...]
    h = h / (1.0 + jnp.exp(-h))
    h = jax.lax.dot(h, w2_ref[...]) + b2_ref[...]
    h = h / (1.0 + jnp.exp(-h))
    out_ref[...] = nd_ref[...] + x2 * h


def _gather_body(qs_hbm, d_hbm, src_hbm, dstg_hbm, qs_out, d_out,
                 idx_s, idx_d, qs_rows, d_rows, gsem, csem):
    gj = src_hbm.shape[1]
    c = lax.axis_index("c")
    s = lax.axis_index("s")
    wid = c * _NS + s
    base = wid * (gj * _CHUNK)
    pltpu.sync_copy(src_hbm.at[wid], idx_s)
    pltpu.sync_copy(dstg_hbm.at[wid], idx_d)

    def fire(j):
        sl = j % 2
        pltpu.async_copy(qs_hbm.at[idx_s.at[j]],
                         qs_rows.at[pl.ds(sl * _CHUNK, _CHUNK)], gsem.at[sl])
        pltpu.async_copy(d_hbm.at[idx_d.at[j]],
                         d_rows.at[pl.ds(sl * _CHUNK, _CHUNK)], gsem.at[sl])

    def drain_gather(j):
        sl = j % 2
        pltpu.make_async_copy(qs_hbm.at[idx_s.at[j]],
                              qs_rows.at[pl.ds(sl * _CHUNK, _CHUNK)],
                              gsem.at[sl]).wait()
        pltpu.make_async_copy(d_hbm.at[idx_d.at[j]],
                              d_rows.at[pl.ds(sl * _CHUNK, _CHUNK)],
                              gsem.at[sl]).wait()

    def fire_out(j):
        sl = j % 2
        row = base + j * _CHUNK
        pltpu.async_copy(qs_rows.at[pl.ds(sl * _CHUNK, _CHUNK)],
                         qs_out.at[pl.ds(row, _CHUNK)], csem.at[sl])
        pltpu.async_copy(d_rows.at[pl.ds(sl * _CHUNK, _CHUNK)],
                         d_out.at[pl.ds(row, _CHUNK)], csem.at[sl])

    def drain_out(j):
        sl = j % 2
        row = base + j * _CHUNK
        pltpu.make_async_copy(qs_rows.at[pl.ds(sl * _CHUNK, _CHUNK)],
                              qs_out.at[pl.ds(row, _CHUNK)], csem.at[sl]).wait()
        pltpu.make_async_copy(d_rows.at[pl.ds(sl * _CHUNK, _CHUNK)],
                              d_out.at[pl.ds(row, _CHUNK)], csem.at[sl]).wait()

    fire(0)
    for j in range(gj):
        if j + 1 < gj:
            if j + 1 >= 2:
                drain_out(j - 1)
            fire(j + 1)
        drain_gather(j)
        fire_out(j)
    drain_out(gj - 1)


def _scatter_body(evx_hbm, dsts_hbm, zeros_hbm, out_hbm,
                  idx_v, idx2_v, ev_v, acc, sem):
    sj = dsts_hbm.shape[1]
    c = lax.axis_index("c")
    s = lax.axis_index("s")
    half = c * _HALF
    pltpu.sync_copy(zeros_hbm, acc.at[pl.ds(s * _STRIPE, _STRIPE)])
    pltpu.sync_copy(dsts_hbm.at[s], idx_v)

    def tbody(j, carry):
        for k in range(_CHUNK // 16):
            v = idx_v[j, pl.ds(k * 16, 16)]
            loc = v - half
            ok = (loc >= 0) & (loc < _HALF)
            idx2_v[j, pl.ds(k * 16, 16)] = jnp.where(ok, loc, _HALF)
        return carry

    lax.fori_loop(0, sj, tbody, 0)
    plsc.subcore_barrier()

    base = s * (sj * _CHUNK)

    def body(j, carry):
        pltpu.sync_copy(evx_hbm.at[pl.ds(base + j * _CHUNK, _CHUNK)], ev_v)
        pltpu.sync_copy(ev_v, acc.at[idx2_v.at[j]], add=True)
        return carry

    lax.fori_loop(0, sj, body, 0)
    plsc.subcore_barrier()

    @pl.when(s < _NS - 1)
    def _():
        pltpu.sync_copy(acc.at[pl.ds(s * _STRIPE, _STRIPE)],
                        out_hbm.at[pl.ds(half + s * _STRIPE, _STRIPE)])

    @pl.when(s == _NS - 1)
    def _():
        last = _HALF - (_NS - 1) * _STRIPE
        pltpu.sync_copy(acc.at[pl.ds((_NS - 1) * _STRIPE, last)],
                        out_hbm.at[pl.ds(half + (_NS - 1) * _STRIPE, last)])


def kernel(node, rbf, rsh, edge_index, Wq, Wsrc, Wdst, Wrbf, brbf, Wkv, Wmsg,
           ln_w, ln_b, Wmlp1, bmlp1, Wmlp2, bmlp2):
    n = node.shape[0]
    E = edge_index.shape[1]
    f32 = jnp.float32
    src_idx = edge_index[0]
    dst_idx = edge_index[1]

    epad = ((E + _NW * _CHUNK - 1) // (_NW * _CHUNK)) * (_NW * _CHUNK)
    gj = epad // _NW // _CHUNK
    sj = epad // _NS // _CHUNK
    src3 = jnp.pad(src_idx, (0, epad - E)).reshape(_NW, gj, _CHUNK)
    dstg3 = jnp.pad(dst_idx, (0, epad - E)).reshape(_NW, gj, _CHUNK)
    dsts3 = jnp.pad(dst_idx, (0, epad - E),
                    constant_values=n).reshape(_NS, sj, _CHUNK)
    rbf_p = jnp.pad(rbf, ((0, epad - E), (0, 0)))
    rsh_p = jnp.pad(rsh, ((0, epad - E), (0, 0)))

    Wcat = jnp.concatenate([Wq, Wsrc, Wdst], axis=1)
    Bn = 1000 if n % 1000 == 0 else n
    proj = pl.pallas_call(
        _proj_body,
        grid=(n // Bn,),
        in_specs=[
            pl.BlockSpec((Bn, D_NODE), lambda i: (i, 0)),
            pl.BlockSpec((D_NODE, 320), lambda i: (0, 0)),
        ],
        out_specs=pl.BlockSpec((Bn, 320), lambda i: (i, 0)),
        out_shape=jax.ShapeDtypeStruct((n, 320), f32),
    )(node, Wcat)
    qs_table = proj[:, :_QS]
    d_table = proj[:, _QS:]

    gather = pl.kernel(
        _gather_body,
        out_type=[jax.ShapeDtypeStruct((epad, _QS), f32),
                  jax.ShapeDtypeStruct((epad, D_MSG), f32)],
        mesh=plsc.VectorSubcoreMesh(core_axis_name="c", subcore_axis_name="s"),
        compiler_params=pltpu.CompilerParams(use_tc_tiling_on_sc=False),
        scratch_types=[
            pltpu.VMEM((gj, _CHUNK), jnp.int32),
            pltpu.VMEM((gj, _CHUNK), jnp.int32),
            pltpu.VMEM((2 * _CHUNK, _QS), f32),
            pltpu.VMEM((2 * _CHUNK, D_MSG), f32),
            pltpu.SemaphoreType.DMA((2,)),
            pltpu.SemaphoreType.DMA((2,)),
        ],
    )
    qs_g, d_g = gather(qs_table, d_table, src3, dstg3)

    idx5 = jnp.arange(512, dtype=jnp.int32)
    rep32 = (idx5[None, :] // 16 == jnp.arange(32, dtype=jnp.int32)[:, None]).astype(jnp.bfloat16)
    tile16 = (idx5[None, :] % 16 == jnp.arange(16, dtype=jnp.int32)[:, None]).astype(jnp.bfloat16)
    idx256 = jnp.arange(256, dtype=jnp.int32)
    shead = (idx256[:, None] // 32 == jnp.arange(8, dtype=jnp.int32)[None, :]).astype(f32)
    rep8 = (idx256[None, :] // 32 == jnp.arange(8, dtype=jnp.int32)[:, None]).astype(f32)

    Eb = 1024
    evx = pl.pallas_call(
        _edge_body,
        grid=(epad // Eb,),
        in_specs=[
            pl.BlockSpec((Eb, _QS), lambda i: (i, 0)),
            pl.BlockSpec((Eb, D_MSG), lambda i: (i, 0)),
            pl.BlockSpec((Eb, N_BASIS), lambda i: (i, 0)),
            pl.BlockSpec((Eb, D_EDGE), lambda i: (i, 0)),
            pl.BlockSpec((N_BASIS, D_EDGE), lambda i: (0, 0)),
            pl.BlockSpec((1, D_EDGE), lambda i: (0, 0)),
            pl.BlockSpec((512, 512), lambda i: (0, 0)),
            pl.BlockSpec((32, 512), lambda i: (0, 0)),
            pl.BlockSpec((16, 512), lambda i: (0, 0)),
            pl.BlockSpec((256, 8), lambda i: (0, 0)),
            pl.BlockSpec((8, 256), lambda i: (0, 0)),
        ],
        out_specs=[pl.BlockSpec((Eb, 128), lambda i: (i, 0)),
                   pl.BlockSpec((Eb, 144), lambda i: (i, 0))],
        out_shape=[jax.ShapeDtypeStruct((epad, 128), f32),
                   jax.ShapeDtypeStruct((epad, 144), f32)],
    )(qs_g, d_g, rbf_p, rsh_p, Wrbf, brbf.reshape(1, -1), Wkv,
      rep32, tile16, shead, rep8)
    evx_lo, evx_hi = evx

    def make_scatter(w):
        return pl.kernel(
            _scatter_body,
            out_type=jax.ShapeDtypeStruct((n, w), f32),
            mesh=plsc.VectorSubcoreMesh(core_axis_name="c",
                                        subcore_axis_name="s"),
            compiler_params=pltpu.CompilerParams(use_tc_tiling_on_sc=False),
            scratch_types=[
                pltpu.VMEM((sj, _CHUNK), jnp.int32),
                pltpu.VMEM((sj, _CHUNK), jnp.int32),
                pltpu.VMEM((_CHUNK, w), f32),
                pltpu.VMEM_SHARED((_NS * _STRIPE, w), f32),
                pltpu.SemaphoreType.DMA,
            ],
        )

    acc_lo = make_scatter(128)(evx_lo, dsts3, jnp.zeros((_STRIPE, 128), f32))
    acc_hi = make_scatter(144)(evx_hi, dsts3, jnp.zeros((_STRIPE, 144), f32))

    out = pl.pallas_call(
        _node_body,
        grid=(n // Bn,),
        in_specs=[
            pl.BlockSpec((Bn, D_NODE), lambda i: (i, 0)),
            pl.BlockSpec((Bn, 128), lambda i: (i, 0)),
            pl.BlockSpec((Bn, 144), lambda i: (i, 0)),
            pl.BlockSpec((D_NODE, D_NODE), lambda i: (0, 0)),
            pl.BlockSpec((1, D_NODE), lambda i: (0, 0)),
            pl.BlockSpec((1, D_NODE), lambda i: (0, 0)),
            pl.BlockSpec((D_NODE, D_NODE), lambda i: (0, 0)),
            pl.BlockSpec((1, D_NODE), lambda i: (0, 0)),
            pl.BlockSpec((D_NODE, D_NODE), lambda i: (0, 0)),
            pl.BlockSpec((1, D_NODE), lambda i: (0, 0)),
            pl.BlockSpec((8, 256), lambda i: (0, 0)),
        ],
        out_specs=pl.BlockSpec((Bn, D_NODE), lambda i: (i, 0)),
        out_shape=jax.ShapeDtypeStruct((n, D_NODE), f32),
    )(node, acc_lo, acc_hi, Wmsg, ln_w.reshape(1, -1), ln_b.reshape(1, -1),
      Wmlp1, bmlp1.reshape(1, -1), Wmlp2, bmlp2.reshape(1, -1), rep8)
    return out


# speedup vs baseline: 10.3652x; 1.1136x over previous
import functools

import jax
import jax.numpy as jnp
import numpy as np
from jax import lax
from jax.experimental import pallas as pl
from jax.experimental.pallas import tpu as pltpu
from jax.experimental.pallas import tpu_sc as plsc

D_NODE = 256
D_EDGE = 16
N_BASIS = 16
D_MSG = 32
N_HEADS = 8
SCALE = 1.0 / np.sqrt(D_MSG)

_NC, _NS = 2, 16
_NW = _NC * _NS
_CHUNK = 128
_QS = D_NODE + D_MSG
_ROW = 272
_HALF = 5000
_STRIPE = 320


def _rep_exact(x, sel_bf16):
    f32 = jnp.float32
    bf16 = jnp.bfloat16
    xh = x.astype(bf16)
    r1 = x - xh.astype(f32)
    xm = r1.astype(bf16)
    xl = (r1 - xm.astype(f32)).astype(bf16)
    out = jax.lax.dot(xh, sel_bf16, preferred_element_type=f32)
    out = out + jax.lax.dot(xm, sel_bf16, preferred_element_type=f32)
    return out + jax.lax.dot(xl, sel_bf16, preferred_element_type=f32)


def _proj_body(node_ref, w_ref, out_ref):
    out_ref[...] = jax.lax.dot(node_ref[...], w_ref[...])


def _edge_body(qs_ref, dg_ref, rbf_ref, rsh_ref, wrbf_ref, brbf_ref,
               wkv_ref, rep32_ref, tile16_ref, shead_ref, rep8_ref,
               lo_ref, hi_ref):
    hi = jax.lax.Precision.HIGHEST
    qg = qs_ref[:, :D_NODE]
    s = qs_ref[:, D_NODE:]
    a = s + dg_ref[...]
    edge = rsh_ref[...] * (
        jax.lax.dot(rbf_ref[...], wrbf_ref[...]) + brbf_ref[...])
    arep = _rep_exact(a, rep32_ref[...])
    etile = _rep_exact(edge, tile16_ref[...])
    coupled = arep * etile
    kv = jax.lax.dot(coupled, wkv_ref[...])
    kfl = kv[:, :D_NODE]
    vfl = kv[:, D_NODE:]
    proj = _rep_exact(qg * kfl, shead_ref[...])
    ex = jnp.exp(proj * SCALE)
    exrep = _rep_exact(ex, rep8_ref[...])
    ev = exrep * vfl
    zer = jnp.zeros_like(ex)
    lo_ref[...] = ev[:, :128]
    hi_ref[...] = jnp.concatenate([ev[:, 128:], ex, zer], axis=1)


def _node_body(nd_ref, lo_ref, hi_ref, wmsg_ref, lnw_ref, lnb_ref,
               w1_ref, b1_ref, w2_ref, b2_ref, rep8_ref, out_ref):
    hi = jax.lax.Precision.HIGHEST
    num = jnp.concatenate([lo_ref[...], hi_ref[:, :128]], axis=1)
    den = hi_ref[:, 128:128 + N_HEADS]
    denrep = jax.lax.dot(den, rep8_ref[...], precision=hi)
    msg = num / (denrep + 1e-16)
    message = jax.lax.dot(msg, wmsg_ref[...])
    x0 = jnp.abs(message)
    mu = jnp.mean(x0, axis=-1, keepdims=True)
    var = jnp.mean((x0 - mu) ** 2, axis=-1, keepdims=True)
    x1 = (x0 - mu) * jax.lax.rsqrt(var + 1e-5) * lnw_ref[...] + lnb_ref[...]
    x2 = message / (x0 + 1e-6)
    h = jax.lax.dot(x1, w1_ref[...]) + b1_ref[...]
    h = h / (1.0 + jnp.exp(-h))
    h = jax.lax.dot(h, w2_ref[...]) + b2_ref[...]
    h = h / (1.0 + jnp.exp(-h))
    out_ref[...] = nd_ref[...] + x2 * h


def _gather_body(qs_hbm, d_hbm, src_hbm, dstg_hbm, qs_out, d_out,
                 idx_s, idx_d, qs_rows, d_rows, gsem, csem):
    gj = src_hbm.shape[1]
    c = lax.axis_index("c")
    s = lax.axis_index("s")
    wid = c * _NS + s
    base = wid * (gj * _CHUNK)
    pltpu.sync_copy(src_hbm.at[wid], idx_s)
    pltpu.sync_copy(dstg_hbm.at[wid], idx_d)

    def fire(j):
        sl = j % 2
        pltpu.async_copy(qs_hbm.at[idx_s.at[j]],
                         qs_rows.at[pl.ds(sl * _CHUNK, _CHUNK)], gsem.at[sl])
        pltpu.async_copy(d_hbm.at[idx_d.at[j]],
                         d_rows.at[pl.ds(sl * _CHUNK, _CHUNK)], gsem.at[sl])

    def drain_gather(j):
        sl = j % 2
        pltpu.make_async_copy(qs_hbm.at[idx_s.at[j]],
                              qs_rows.at[pl.ds(sl * _CHUNK, _CHUNK)],
                              gsem.at[sl]).wait()
        pltpu.make_async_copy(d_hbm.at[idx_d.at[j]],
                              d_rows.at[pl.ds(sl * _CHUNK, _CHUNK)],
                              gsem.at[sl]).wait()

    def fire_out(j):
        sl = j % 2
        row = base + j * _CHUNK
        pltpu.async_copy(qs_rows.at[pl.ds(sl * _CHUNK, _CHUNK)],
                         qs_out.at[pl.ds(row, _CHUNK)], csem.at[sl])
        pltpu.async_copy(d_rows.at[pl.ds(sl * _CHUNK, _CHUNK)],
                         d_out.at[pl.ds(row, _CHUNK)], csem.at[sl])

    def drain_out(j):
        sl = j % 2
        row = base + j * _CHUNK
        pltpu.make_async_copy(qs_rows.at[pl.ds(sl * _CHUNK, _CHUNK)],
                              qs_out.at[pl.ds(row, _CHUNK)], csem.at[sl]).wait()
        pltpu.make_async_copy(d_rows.at[pl.ds(sl * _CHUNK, _CHUNK)],
                              d_out.at[pl.ds(row, _CHUNK)], csem.at[sl]).wait()

    fire(0)
    for j in range(gj):
        if j + 1 < gj:
            if j + 1 >= 2:
                drain_out(j - 1)
            fire(j + 1)
        drain_gather(j)
        fire_out(j)
    drain_out(gj - 1)


def _scatter_body(evx_hbm, dsts_hbm, zeros_hbm, out_hbm,
                  idx_v, idx2_v, ev_v, acc, sem):
    sj = dsts_hbm.shape[1]
    c = lax.axis_index("c")
    s = lax.axis_index("s")
    half = c * _HALF
    pltpu.sync_copy(zeros_hbm, acc.at[pl.ds(s * _STRIPE, _STRIPE)])
    pltpu.sync_copy(dsts_hbm.at[s], idx_v)

    def tbody(j, carry):
        for k in range(_CHUNK // 16):
            v = idx_v[j, pl.ds(k * 16, 16)]
            loc = v - half
            ok = (loc >= 0) & (loc < _HALF)
            idx2_v[j, pl.ds(k * 16, 16)] = jnp.where(ok, loc, _HALF)
        return carry

    lax.fori_loop(0, sj, tbody, 0)
    plsc.subcore_barrier()

    base = s * (sj * _CHUNK)

    def body(j, carry):
        pltpu.sync_copy(evx_hbm.at[pl.ds(base + j * _CHUNK, _CHUNK)], ev_v)
        pltpu.sync_copy(ev_v, acc.at[idx2_v.at[j]], add=True)
        return carry

    lax.fori_loop(0, sj, body, 0)
    plsc.subcore_barrier()

    @pl.when(s < _NS - 1)
    def _():
        pltpu.sync_copy(acc.at[pl.ds(s * _STRIPE, _STRIPE)],
                        out_hbm.at[pl.ds(half + s * _STRIPE, _STRIPE)])

    @pl.when(s == _NS - 1)
    def _():
        last = _HALF - (_NS - 1) * _STRIPE
        pltpu.sync_copy(acc.at[pl.ds((_NS - 1) * _STRIPE, last)],
                        out_hbm.at[pl.ds(half + (_NS - 1) * _STRIPE, last)])


def kernel(node, rbf, rsh, edge_index, Wq, Wsrc, Wdst, Wrbf, brbf, Wkv, Wmsg,
           ln_w, ln_b, Wmlp1, bmlp1, Wmlp2, bmlp2):
    n = node.shape[0]
    E = edge_index.shape[1]
    f32 = jnp.float32
    src_idx = edge_index[0]
    dst_idx = edge_index[1]

    epad = ((E + _NW * _CHUNK - 1) // (_NW * _CHUNK)) * (_NW * _CHUNK)
    gj = epad // _NW // _CHUNK
    sj = epad // _NS // _CHUNK
    src3 = jnp.pad(src_idx, (0, epad - E)).reshape(_NW, gj, _CHUNK)
    dstg3 = jnp.pad(dst_idx, (0, epad - E)).reshape(_NW, gj, _CHUNK)
    dsts3 = jnp.pad(dst_idx, (0, epad - E),
                    constant_values=n).reshape(_NS, sj, _CHUNK)
    rbf_p = jnp.pad(rbf, ((0, epad - E), (0, 0)))
    rsh_p = jnp.pad(rsh, ((0, epad - E), (0, 0)))

    Wcat = jnp.concatenate([Wq, Wsrc, Wdst], axis=1)
    Bn = 1000 if n % 1000 == 0 else n
    proj = pl.pallas_call(
        _proj_body,
        grid=(n // Bn,),
        in_specs=[
            pl.BlockSpec((Bn, D_NODE), lambda i: (i, 0)),
            pl.BlockSpec((D_NODE, 320), lambda i: (0, 0)),
        ],
        out_specs=pl.BlockSpec((Bn, 320), lambda i: (i, 0)),
        out_shape=jax.ShapeDtypeStruct((n, 320), f32),
    )(node, Wcat)
    qs_table = proj[:, :_QS]
    d_table = proj[:, _QS:]

    gather = pl.kernel(
        _gather_body,
        out_type=[jax.ShapeDtypeStruct((epad, _QS), f32),
                  jax.ShapeDtypeStruct((epad, D_MSG), f32)],
        mesh=plsc.VectorSubcoreMesh(core_axis_name="c", subcore_axis_name="s"),
        compiler_params=pltpu.CompilerParams(use_tc_tiling_on_sc=False),
        scratch_types=[
            pltpu.VMEM((gj, _CHUNK), jnp.int32),
            pltpu.VMEM((gj, _CHUNK), jnp.int32),
            pltpu.VMEM((2 * _CHUNK, _QS), f32),
            pltpu.VMEM((2 * _CHUNK, D_MSG), f32),
            pltpu.SemaphoreType.DMA((2,)),
            pltpu.SemaphoreType.DMA((2,)),
        ],
    )
    qs_g, d_g = gather(qs_table, d_table, src3, dstg3)

    idx5 = jnp.arange(512, dtype=jnp.int32)
    rep32 = (idx5[None, :] // 16 == jnp.arange(32, dtype=jnp.int32)[:, None]).astype(jnp.bfloat16)
    tile16 = (idx5[None, :] % 16 == jnp.arange(16, dtype=jnp.int32)[:, None]).astype(jnp.bfloat16)
    idx256 = jnp.arange(256, dtype=jnp.int32)
    shead = (idx256[:, None] // 32 == jnp.arange(8, dtype=jnp.int32)[None, :]).astype(jnp.bfloat16)
    rep8 = (idx256[None, :] // 32 == jnp.arange(8, dtype=jnp.int32)[:, None]).astype(jnp.bfloat16)
    rep8f = rep8.astype(f32)

    Eb = 1024
    evx = pl.pallas_call(
        _edge_body,
        grid=(epad // Eb,),
        in_specs=[
            pl.BlockSpec((Eb, _QS), lambda i: (i, 0)),
            pl.BlockSpec((Eb, D_MSG), lambda i: (i, 0)),
            pl.BlockSpec((Eb, N_BASIS), lambda i: (i, 0)),
            pl.BlockSpec((Eb, D_EDGE), lambda i: (i, 0)),
            pl.BlockSpec((N_BASIS, D_EDGE), lambda i: (0, 0)),
            pl.BlockSpec((1, D_EDGE), lambda i: (0, 0)),
            pl.BlockSpec((512, 512), lambda i: (0, 0)),
            pl.BlockSpec((32, 512), lambda i: (0, 0)),
            pl.BlockSpec((16, 512), lambda i: (0, 0)),
            pl.BlockSpec((256, 8), lambda i: (0, 0)),
            pl.BlockSpec((8, 256), lambda i: (0, 0)),
        ],
        out_specs=[pl.BlockSpec((Eb, 128), lambda i: (i, 0)),
                   pl.BlockSpec((Eb, 144), lambda i: (i, 0))],
        out_shape=[jax.ShapeDtypeStruct((epad, 128), f32),
                   jax.ShapeDtypeStruct((epad, 144), f32)],
    )(qs_g, d_g, rbf_p, rsh_p, Wrbf, brbf.reshape(1, -1), Wkv,
      rep32, tile16, shead, rep8)
    evx_lo, evx_hi = evx

    def make_scatter(w):
        return pl.kernel(
            _scatter_body,
            out_type=jax.ShapeDtypeStruct((n, w), f32),
            mesh=plsc.VectorSubcoreMesh(core_axis_name="c",
                                        subcore_axis_name="s"),
            compiler_params=pltpu.CompilerParams(use_tc_tiling_on_sc=False),
            scratch_types=[
                pltpu.VMEM((sj, _CHUNK), jnp.int32),
                pltpu.VMEM((sj, _CHUNK), jnp.int32),
                pltpu.VMEM((_CHUNK, w), f32),
                pltpu.VMEM_SHARED((_NS * _STRIPE, w), f32),
                pltpu.SemaphoreType.DMA,
            ],
        )

    acc_lo = make_scatter(128)(evx_lo, dsts3, jnp.zeros((_STRIPE, 128), f32))
    acc_hi = make_scatter(144)(evx_hi, dsts3, jnp.zeros((_STRIPE, 144), f32))

    out = pl.pallas_call(
        _node_body,
        grid=(n // Bn,),
        in_specs=[
            pl.BlockSpec((Bn, D_NODE), lambda i: (i, 0)),
            pl.BlockSpec((Bn, 128), lambda i: (i, 0)),
            pl.BlockSpec((Bn, 144), lambda i: (i, 0)),
            pl.BlockSpec((D_NODE, D_NODE), lambda i: (0, 0)),
            pl.BlockSpec((1, D_NODE), lambda i: (0, 0)),
            pl.BlockSpec((1, D_NODE), lambda i: (0, 0)),
            pl.BlockSpec((D_NODE, D_NODE), lambda i: (0, 0)),
            pl.BlockSpec((1, D_NODE), lambda i: (0, 0)),
            pl.BlockSpec((D_NODE, D_NODE), lambda i: (0, 0)),
            pl.BlockSpec((1, D_NODE), lambda i: (0, 0)),
            pl.BlockSpec((8, 256), lambda i: (0, 0)),
        ],
        out_specs=pl.BlockSpec((Bn, D_NODE), lambda i: (i, 0)),
        out_shape=jax.ShapeDtypeStruct((n, D_NODE), f32),
    )(node, acc_lo, acc_hi, Wmsg, ln_w.reshape(1, -1), ln_b.reshape(1, -1),
      Wmlp1, bmlp1.reshape(1, -1), Wmlp2, bmlp2.reshape(1, -1), rep8f)
    return out


# speedup vs baseline: 11.0941x; 1.0703x over previous
import functools

import jax
import jax.numpy as jnp
import numpy as np
from jax import lax
from jax.experimental import pallas as pl
from jax.experimental.pallas import tpu as pltpu
from jax.experimental.pallas import tpu_sc as plsc

D_NODE = 256
D_EDGE = 16
N_BASIS = 16
D_MSG = 32
N_HEADS = 8
SCALE = 1.0 / np.sqrt(D_MSG)

_NC, _NS = 2, 16
_NW = _NC * _NS
_CHUNK = 128
_QS = D_NODE + D_MSG
_ROW = 272
_NNODE = 10000
_STRIPE = 640


def _rep_exact(x, sel_bf16):
    f32 = jnp.float32
    bf16 = jnp.bfloat16
    xh = x.astype(bf16)
    r1 = x - xh.astype(f32)
    xm = r1.astype(bf16)
    xl = (r1 - xm.astype(f32)).astype(bf16)
    out = jax.lax.dot(xh, sel_bf16, preferred_element_type=f32)
    out = out + jax.lax.dot(xm, sel_bf16, preferred_element_type=f32)
    return out + jax.lax.dot(xl, sel_bf16, preferred_element_type=f32)


def _proj_body(node_ref, w_ref, out_ref):
    out_ref[...] = jax.lax.dot(node_ref[...], w_ref[...])


def _edge_body(qs_ref, dg_ref, rbf_ref, rsh_ref, wrbf_ref, brbf_ref,
               wkv_ref, rep32_ref, tile16_ref, shead_ref, rep8_ref,
               p0_ref, p1_ref, p2_ref):
    hi = jax.lax.Precision.HIGHEST
    qg = qs_ref[:, :D_NODE]
    s = qs_ref[:, D_NODE:]
    a = s + dg_ref[...]
    edge = rsh_ref[...] * (
        jax.lax.dot(rbf_ref[...], wrbf_ref[...]) + brbf_ref[...])
    arep = _rep_exact(a, rep32_ref[...])
    etile = _rep_exact(edge, tile16_ref[...])
    coupled = arep * etile
    kv = jax.lax.dot(coupled, wkv_ref[...])
    kfl = kv[:, :D_NODE]
    vfl = kv[:, D_NODE:]
    proj = _rep_exact(qg * kfl, shead_ref[...])
    ex = jnp.exp(proj * SCALE)
    exrep = _rep_exact(ex, rep8_ref[...])
    ev = exrep * vfl
    zer = jnp.zeros_like(ex)
    p0_ref[...] = ev[:, :112]
    p1_ref[...] = ev[:, 112:224]
    p2_ref[...] = jnp.concatenate([ev[:, 224:], ex, zer], axis=1)


def _node_body(nd_ref, p0_ref, p1_ref, p2_ref, wmsg_ref, lnw_ref, lnb_ref,
               w1_ref, b1_ref, w2_ref, b2_ref, rep8_ref, out_ref):
    hi = jax.lax.Precision.HIGHEST
    p0 = p0_ref[0] + p0_ref[1]
    p1 = p1_ref[0] + p1_ref[1]
    p2 = p2_ref[0] + p2_ref[1]
    num = jnp.concatenate([p0, p1, p2[:, :32]], axis=1)
    den = p2[:, 32:32 + N_HEADS]
    denrep = jax.lax.dot(den, rep8_ref[...], precision=hi)
    msg = num / (denrep + 1e-16)
    message = jax.lax.dot(msg, wmsg_ref[...])
    x0 = jnp.abs(message)
    mu = jnp.mean(x0, axis=-1, keepdims=True)
    var = jnp.mean((x0 - mu) ** 2, axis=-1, keepdims=True)
    x1 = (x0 - mu) * jax.lax.rsqrt(var + 1e-5) * lnw_ref[...] + lnb_ref[...]
    x2 = message / (x0 + 1e-6)
    h = jax.lax.dot(x1, w1_ref[...]) + b1_ref[...]
    h = h / (1.0 + jnp.exp(-h))
    h = jax.lax.dot(h, w2_ref[...]) + b2_ref[...]
    h = h / (1.0 + jnp.exp(-h))
    out_ref[...] = nd_ref[...] + x2 * h


def _gather_body(qs_hbm, d_hbm, src_hbm, dstg_hbm, qs_out, d_out,
                 idx_s, idx_d, qs_rows, d_rows, gsem, csem):
    gj = src_hbm.shape[1]
    c = lax.axis_index("c")
    s = lax.axis_index("s")
    wid = c * _NS + s
    base = wid * (gj * _CHUNK)
    pltpu.sync_copy(src_hbm.at[wid], idx_s)
    pltpu.sync_copy(dstg_hbm.at[wid], idx_d)

    def fire(j):
        sl = j % 2
        pltpu.async_copy(qs_hbm.at[idx_s.at[j]],
                         qs_rows.at[pl.ds(sl * _CHUNK, _CHUNK)], gsem.at[sl])
        pltpu.async_copy(d_hbm.at[idx_d.at[j]],
                         d_rows.at[pl.ds(sl * _CHUNK, _CHUNK)], gsem.at[sl])

    def drain_gather(j):
        sl = j % 2
        pltpu.make_async_copy(qs_hbm.at[idx_s.at[j]],
                              qs_rows.at[pl.ds(sl * _CHUNK, _CHUNK)],
                              gsem.at[sl]).wait()
        pltpu.make_async_copy(d_hbm.at[idx_d.at[j]],
                              d_rows.at[pl.ds(sl * _CHUNK, _CHUNK)],
                              gsem.at[sl]).wait()

    def fire_out(j):
        sl = j % 2
        row = base + j * _CHUNK
        pltpu.async_copy(qs_rows.at[pl.ds(sl * _CHUNK, _CHUNK)],
                         qs_out.at[pl.ds(row, _CHUNK)], csem.at[sl])
        pltpu.async_copy(d_rows.at[pl.ds(sl * _CHUNK, _CHUNK)],
                         d_out.at[pl.ds(row, _CHUNK)], csem.at[sl])

    def drain_out(j):
        sl = j % 2
        row = base + j * _CHUNK
        pltpu.make_async_copy(qs_rows.at[pl.ds(sl * _CHUNK, _CHUNK)],
                              qs_out.at[pl.ds(row, _CHUNK)], csem.at[sl]).wait()
        pltpu.make_async_copy(d_rows.at[pl.ds(sl * _CHUNK, _CHUNK)],
                              d_out.at[pl.ds(row, _CHUNK)], csem.at[sl]).wait()

    fire(0)
    for j in range(gj):
        if j + 1 < gj:
            if j + 1 >= 2:
                drain_out(j - 1)
            fire(j + 1)
        drain_gather(j)
        fire_out(j)
    drain_out(gj - 1)


def _scatter_body(evx_hbm, dsts_hbm, zeros_hbm, out_hbm,
                  idx_v, ev_v, acc, sem):
    sj = dsts_hbm.shape[1]
    c = lax.axis_index("c")
    s = lax.axis_index("s")
    n_acc = _NS * _STRIPE
    pltpu.sync_copy(zeros_hbm, acc.at[pl.ds(s * _STRIPE, _STRIPE)])
    wid = c * _NS + s
    pltpu.sync_copy(dsts_hbm.at[wid], idx_v)
    plsc.subcore_barrier()

    base = wid * (sj * _CHUNK)

    def body(j, carry):
        pltpu.sync_copy(evx_hbm.at[pl.ds(base + j * _CHUNK, _CHUNK)], ev_v)
        pltpu.sync_copy(ev_v, acc.at[idx_v.at[j]], add=True)
        return carry

    lax.fori_loop(0, sj, body, 0)
    plsc.subcore_barrier()

    @pl.when(s < _NS - 1)
    def _():
        pltpu.sync_copy(acc.at[pl.ds(s * _STRIPE, _STRIPE)],
                        out_hbm.at[c, pl.ds(s * _STRIPE, _STRIPE)])

    @pl.when(s == _NS - 1)
    def _():
        last = _NNODE - (_NS - 1) * _STRIPE
        pltpu.sync_copy(acc.at[pl.ds((_NS - 1) * _STRIPE, last)],
                        out_hbm.at[c, pl.ds((_NS - 1) * _STRIPE, last)])


def kernel(node, rbf, rsh, edge_index, Wq, Wsrc, Wdst, Wrbf, brbf, Wkv, Wmsg,
           ln_w, ln_b, Wmlp1, bmlp1, Wmlp2, bmlp2):
    n = node.shape[0]
    E = edge_index.shape[1]
    f32 = jnp.float32
    src_idx = edge_index[0]
    dst_idx = edge_index[1]

    epad = ((E + _NW * _CHUNK - 1) // (_NW * _CHUNK)) * (_NW * _CHUNK)
    gj = epad // _NW // _CHUNK
    sj = epad // _NS // _CHUNK
    src3 = jnp.pad(src_idx, (0, epad - E)).reshape(_NW, gj, _CHUNK)
    dstg3 = jnp.pad(dst_idx, (0, epad - E)).reshape(_NW, gj, _CHUNK)
    dsts3 = jnp.pad(dst_idx, (0, epad - E),
                    constant_values=n).reshape(_NS, sj, _CHUNK)
    rbf_p = jnp.pad(rbf, ((0, epad - E), (0, 0)))
    rsh_p = jnp.pad(rsh, ((0, epad - E), (0, 0)))

    Wcat = jnp.concatenate([Wq, Wsrc, Wdst], axis=1)
    Bn = 1000 if n % 1000 == 0 else n
    proj = pl.pallas_call(
        _proj_body,
        grid=(n // Bn,),
        in_specs=[
            pl.BlockSpec((Bn, D_NODE), lambda i: (i, 0)),
            pl.BlockSpec((D_NODE, 320), lambda i: (0, 0)),
        ],
        out_specs=pl.BlockSpec((Bn, 320), lambda i: (i, 0)),
        out_shape=jax.ShapeDtypeStruct((n, 320), f32),
    )(node, Wcat)
    qs_table = proj[:, :_QS]
    d_table = proj[:, _QS:]

    gather = pl.kernel(
        _gather_body,
        out_type=[jax.ShapeDtypeStruct((epad, _QS), f32),
                  jax.ShapeDtypeStruct((epad, D_MSG), f32)],
        mesh=plsc.VectorSubcoreMesh(core_axis_name="c", subcore_axis_name="s"),
        compiler_params=pltpu.CompilerParams(use_tc_tiling_on_sc=False),
        scratch_types=[
            pltpu.VMEM((gj, _CHUNK), jnp.int32),
            pltpu.VMEM((gj, _CHUNK), jnp.int32),
            pltpu.VMEM((2 * _CHUNK, _QS), f32),
            pltpu.VMEM((2 * _CHUNK, D_MSG), f32),
            pltpu.SemaphoreType.DMA((2,)),
            pltpu.SemaphoreType.DMA((2,)),
        ],
    )
    qs_g, d_g = gather(qs_table, d_table, src3, dstg3)

    idx5 = jnp.arange(512, dtype=jnp.int32)
    rep32 = (idx5[None, :] // 16 == jnp.arange(32, dtype=jnp.int32)[:, None]).astype(jnp.bfloat16)
    tile16 = (idx5[None, :] % 16 == jnp.arange(16, dtype=jnp.int32)[:, None]).astype(jnp.bfloat16)
    idx256 = jnp.arange(256, dtype=jnp.int32)
    shead = (idx256[:, None] // 32 == jnp.arange(8, dtype=jnp.int32)[None, :]).astype(jnp.bfloat16)
    rep8 = (idx256[None, :] // 32 == jnp.arange(8, dtype=jnp.int32)[:, None]).astype(jnp.bfloat16)
    rep8f = rep8.astype(f32)

    Eb = 1024
    evx = pl.pallas_call(
        _edge_body,
        grid=(epad // Eb,),
        in_specs=[
            pl.BlockSpec((Eb, _QS), lambda i: (i, 0)),
            pl.BlockSpec((Eb, D_MSG), lambda i: (i, 0)),
            pl.BlockSpec((Eb, N_BASIS), lambda i: (i, 0)),
            pl.BlockSpec((Eb, D_EDGE), lambda i: (i, 0)),
            pl.BlockSpec((N_BASIS, D_EDGE), lambda i: (0, 0)),
            pl.BlockSpec((1, D_EDGE), lambda i: (0, 0)),
            pl.BlockSpec((512, 512), lambda i: (0, 0)),
            pl.BlockSpec((32, 512), lambda i: (0, 0)),
            pl.BlockSpec((16, 512), lambda i: (0, 0)),
            pl.BlockSpec((256, 8), lambda i: (0, 0)),
            pl.BlockSpec((8, 256), lambda i: (0, 0)),
        ],
        out_specs=[pl.BlockSpec((Eb, 112), lambda i: (i, 0)),
                   pl.BlockSpec((Eb, 112), lambda i: (i, 0)),
                   pl.BlockSpec((Eb, 48), lambda i: (i, 0))],
        out_shape=[jax.ShapeDtypeStruct((epad, 112), f32),
                   jax.ShapeDtypeStruct((epad, 112), f32),
                   jax.ShapeDtypeStruct((epad, 48), f32)],
    )(qs_g, d_g, rbf_p, rsh_p, Wrbf, brbf.reshape(1, -1), Wkv,
      rep32, tile16, shead, rep8)
    ev_p0, ev_p1, ev_p2 = evx

    sjt = epad // _NW // _CHUNK

    def make_scatter(w):
        return pl.kernel(
            _scatter_body,
            out_type=jax.ShapeDtypeStruct((_NC, n, w), f32),
            mesh=plsc.VectorSubcoreMesh(core_axis_name="c",
                                        subcore_axis_name="s"),
            compiler_params=pltpu.CompilerParams(use_tc_tiling_on_sc=False),
            scratch_types=[
                pltpu.VMEM((sjt, _CHUNK), jnp.int32),
                pltpu.VMEM((_CHUNK, w), f32),
                pltpu.VMEM_SHARED((_NS * _STRIPE, w), f32),
                pltpu.SemaphoreType.DMA,
            ],
        )

    dsts3t = jnp.pad(dst_idx, (0, epad - E),
                     constant_values=n).reshape(_NW, sjt, _CHUNK)
    acc_p0 = make_scatter(112)(ev_p0, dsts3t, jnp.zeros((_STRIPE, 112), f32))
    acc_p1 = make_scatter(112)(ev_p1, dsts3t, jnp.zeros((_STRIPE, 112), f32))
    acc_p2 = make_scatter(48)(ev_p2, dsts3t, jnp.zeros((_STRIPE, 48), f32))

    out = pl.pallas_call(
        _node_body,
        grid=(n // Bn,),
        in_specs=[
            pl.BlockSpec((Bn, D_NODE), lambda i: (i, 0)),
            pl.BlockSpec((_NC, Bn, 112), lambda i: (0, i, 0)),
            pl.BlockSpec((_NC, Bn, 112), lambda i: (0, i, 0)),
            pl.BlockSpec((_NC, Bn, 48), lambda i: (0, i, 0)),
            pl.BlockSpec((D_NODE, D_NODE), lambda i: (0, 0)),
            pl.BlockSpec((1, D_NODE), lambda i: (0, 0)),
            pl.BlockSpec((1, D_NODE), lambda i: (0, 0)),
            pl.BlockSpec((D_NODE, D_NODE), lambda i: (0, 0)),
            pl.BlockSpec((1, D_NODE), lambda i: (0, 0)),
            pl.BlockSpec((D_NODE, D_NODE), lambda i: (0, 0)),
            pl.BlockSpec((1, D_NODE), lambda i: (0, 0)),
            pl.BlockSpec((8, 256), lambda i: (0, 0)),
        ],
        out_specs=pl.BlockSpec((Bn, D_NODE), lambda i: (i, 0)),
        out_shape=jax.ShapeDtypeStruct((n, D_NODE), f32),
    )(node, acc_p0, acc_p1, acc_p2, Wmsg,
      ln_w.reshape(1, -1), ln_b.reshape(1, -1),
      Wmlp1, bmlp1.reshape(1, -1), Wmlp2, bmlp2.reshape(1, -1), rep8f)
    return out


# speedup vs baseline: 11.1411x; 1.0042x over previous
import functools

import jax
import jax.numpy as jnp
import numpy as np
from jax import lax
from jax.experimental import pallas as pl
from jax.experimental.pallas import tpu as pltpu
from jax.experimental.pallas import tpu_sc as plsc

D_NODE = 256
D_EDGE = 16
N_BASIS = 16
D_MSG = 32
N_HEADS = 8
SCALE = 1.0 / np.sqrt(D_MSG)

_NC, _NS = 2, 16
_NW = _NC * _NS
_CHUNK = 128
_QS = D_NODE + D_MSG
_NNODE = 10000
_STRIPE = 640


def _rep_exact(x, sel_bf16):
    f32 = jnp.float32
    bf16 = jnp.bfloat16
    xh = x.astype(bf16)
    r1 = x - xh.astype(f32)
    xm = r1.astype(bf16)
    xl = (r1 - xm.astype(f32)).astype(bf16)
    out = jax.lax.dot(xh, sel_bf16, preferred_element_type=f32)
    out = out + jax.lax.dot(xm, sel_bf16, preferred_element_type=f32)
    return out + jax.lax.dot(xl, sel_bf16, preferred_element_type=f32)


def _proj_body(node_ref, w_ref, out_ref):
    out_ref[...] = jax.lax.dot(node_ref[...], w_ref[...])


def _edge_body(qs_ref, dg_ref, rbf_ref, rsh_ref, wrbf_ref, brbf_ref,
               wkv_ref, rep32_ref, tile16_ref, shead_ref, rep8_ref,
               p0_ref, p1_ref, p2_ref):
    hi = jax.lax.Precision.HIGHEST
    qg = qs_ref[:, :D_NODE]
    s = qs_ref[:, D_NODE:]
    a = s + dg_ref[...]
    edge = rsh_ref[...] * (
        jax.lax.dot(rbf_ref[...], wrbf_ref[...]) + brbf_ref[...])
    arep = _rep_exact(a, rep32_ref[...])
    etile = _rep_exact(edge, tile16_ref[...])
    coupled = arep * etile
    kv = jax.lax.dot(coupled, wkv_ref[...])
    kfl = kv[:, :D_NODE]
    vfl = kv[:, D_NODE:]
    proj = _rep_exact(qg * kfl, shead_ref[...])
    ex = jnp.exp(proj * SCALE)
    exrep = _rep_exact(ex, rep8_ref[...])
    ev = exrep * vfl
    zer = jnp.zeros_like(ex)
    p0_ref[...] = ev[:, :112]
    p1_ref[...] = ev[:, 112:224]
    p2_ref[...] = jnp.concatenate([ev[:, 224:], ex, zer], axis=1)


def _node_body(nd_ref, p0_ref, p1_ref, p2_ref, wmsg_ref, lnw_ref, lnb_ref,
               w1_ref, b1_ref, w2_ref, b2_ref, rep8_ref, out_ref):
    hi = jax.lax.Precision.HIGHEST
    p0 = p0_ref[0] + p0_ref[1]
    p1 = p1_ref[0] + p1_ref[1]
    p2 = p2_ref[0] + p2_ref[1]
    num = jnp.concatenate([p0, p1, p2[:, :32]], axis=1)
    den = p2[:, 32:32 + N_HEADS]
    denrep = jax.lax.dot(den, rep8_ref[...], precision=hi)
    msg = num / (denrep + 1e-16)
    message = jax.lax.dot(msg, wmsg_ref[...])
    x0 = jnp.abs(message)
    mu = jnp.mean(x0, axis=-1, keepdims=True)
    var = jnp.mean((x0 - mu) ** 2, axis=-1, keepdims=True)
    x1 = (x0 - mu) * jax.lax.rsqrt(var + 1e-5) * lnw_ref[...] + lnb_ref[...]
    x2 = message / (x0 + 1e-6)
    h = jax.lax.dot(x1, w1_ref[...]) + b1_ref[...]
    h = h / (1.0 + jnp.exp(-h))
    h = jax.lax.dot(h, w2_ref[...]) + b2_ref[...]
    h = h / (1.0 + jnp.exp(-h))
    out_ref[...] = nd_ref[...] + x2 * h


def _gather_body(qs_hbm, d_hbm, src_hbm, dstg_hbm, qs_out, d_out,
                 idx_s, idx_d, qs_rows, d_rows, gsem, csem):
    gj = src_hbm.shape[1]
    c = lax.axis_index("c")
    s = lax.axis_index("s")
    wid = c * _NS + s
    base = wid * (gj * _CHUNK)
    pltpu.sync_copy(src_hbm.at[wid], idx_s)
    pltpu.sync_copy(dstg_hbm.at[wid], idx_d)

    def fire(j):
        sl = j % 2
        pltpu.async_copy(qs_hbm.at[idx_s.at[j]],
                         qs_rows.at[pl.ds(sl * _CHUNK, _CHUNK)], gsem.at[sl])
        pltpu.async_copy(d_hbm.at[idx_d.at[j]],
                         d_rows.at[pl.ds(sl * _CHUNK, _CHUNK)], gsem.at[sl])

    def drain_gather(j):
        sl = j % 2
        pltpu.make_async_copy(qs_hbm.at[idx_s.at[j]],
                              qs_rows.at[pl.ds(sl * _CHUNK, _CHUNK)],
                              gsem.at[sl]).wait()
        pltpu.make_async_copy(d_hbm.at[idx_d.at[j]],
                              d_rows.at[pl.ds(sl * _CHUNK, _CHUNK)],
                              gsem.at[sl]).wait()

    def fire_out(j):
        sl = j % 2
        row = base + j * _CHUNK
        pltpu.async_copy(qs_rows.at[pl.ds(sl * _CHUNK, _CHUNK)],
                         qs_out.at[pl.ds(row, _CHUNK)], csem.at[sl])
        pltpu.async_copy(d_rows.at[pl.ds(sl * _CHUNK, _CHUNK)],
                         d_out.at[pl.ds(row, _CHUNK)], csem.at[sl])

    def drain_out(j):
        sl = j % 2
        row = base + j * _CHUNK
        pltpu.make_async_copy(qs_rows.at[pl.ds(sl * _CHUNK, _CHUNK)],
                              qs_out.at[pl.ds(row, _CHUNK)], csem.at[sl]).wait()
        pltpu.make_async_copy(d_rows.at[pl.ds(sl * _CHUNK, _CHUNK)],
                              d_out.at[pl.ds(row, _CHUNK)], csem.at[sl]).wait()

    fire(0)
    for j in range(gj):
        if j + 1 < gj:
            if j + 1 >= 2:
                drain_out(j - 1)
            fire(j + 1)
        drain_gather(j)
        fire_out(j)
    drain_out(gj - 1)


def _scatter_body(evx_hbm, dsts_hbm, zeros_hbm, out_hbm,
                  idx_v, ev_v, acc, sem):
    sj = dsts_hbm.shape[1]
    c = lax.axis_index("c")
    s = lax.axis_index("s")
    n_acc = _NS * _STRIPE
    pltpu.sync_copy(zeros_hbm, acc.at[pl.ds(s * _STRIPE, _STRIPE)])
    wid = c * _NS + s
    pltpu.sync_copy(dsts_hbm.at[wid], idx_v)
    plsc.subcore_barrier()

    base = wid * (sj * _CHUNK)

    def body(j, carry):
        pltpu.sync_copy(evx_hbm.at[pl.ds(base + j * _CHUNK, _CHUNK)], ev_v)
        pltpu.sync_copy(ev_v, acc.at[idx_v.at[j]], add=True)
        return carry

    lax.fori_loop(0, sj, body, 0)
    plsc.subcore_barrier()

    @pl.when(s < _NS - 1)
    def _():
        pltpu.sync_copy(acc.at[pl.ds(s * _STRIPE, _STRIPE)],
                        out_hbm.at[c, pl.ds(s * _STRIPE, _STRIPE)])

    @pl.when(s == _NS - 1)
    def _():
        last = _NNODE - (_NS - 1) * _STRIPE
        pltpu.sync_copy(acc.at[pl.ds((_NS - 1) * _STRIPE, last)],
                        out_hbm.at[c, pl.ds((_NS - 1) * _STRIPE, last)])


def kernel(node, rbf, rsh, edge_index, Wq, Wsrc, Wdst, Wrbf, brbf, Wkv, Wmsg,
           ln_w, ln_b, Wmlp1, bmlp1, Wmlp2, bmlp2):
    n = node.shape[0]
    E = edge_index.shape[1]
    f32 = jnp.float32
    src_idx = edge_index[0]
    dst_idx = edge_index[1]

    epad = ((E + _NW * _CHUNK - 1) // (_NW * _CHUNK)) * (_NW * _CHUNK)
    gj = epad // _NW // _CHUNK
    src3 = jnp.pad(src_idx, (0, epad - E)).reshape(_NW, gj, _CHUNK)
    dstg3 = jnp.pad(dst_idx, (0, epad - E)).reshape(_NW, gj, _CHUNK)
    rbf_p = jnp.pad(rbf, ((0, epad - E), (0, 0)))
    rsh_p = jnp.pad(rsh, ((0, epad - E), (0, 0)))

    Wcat = jnp.concatenate([Wq, Wsrc, Wdst], axis=1)
    Bn = 1000 if n % 1000 == 0 else n
    proj = pl.pallas_call(
        _proj_body,
        grid=(n // Bn,),
        in_specs=[
            pl.BlockSpec((Bn, D_NODE), lambda i: (i, 0)),
            pl.BlockSpec((D_NODE, 320), lambda i: (0, 0)),
        ],
        out_specs=pl.BlockSpec((Bn, 320), lambda i: (i, 0)),
        out_shape=jax.ShapeDtypeStruct((n, 320), f32),
    )(node, Wcat)
    qs_table = proj[:, :_QS]
    d_table = proj[:, _QS:]

    gather = pl.kernel(
        _gather_body,
        out_type=[jax.ShapeDtypeStruct((epad, _QS), f32),
                  jax.ShapeDtypeStruct((epad, D_MSG), f32)],
        mesh=plsc.VectorSubcoreMesh(core_axis_name="c", subcore_axis_name="s"),
        compiler_params=pltpu.CompilerParams(use_tc_tiling_on_sc=False),
        scratch_types=[
            pltpu.VMEM((gj, _CHUNK), jnp.int32),
            pltpu.VMEM((gj, _CHUNK), jnp.int32),
            pltpu.VMEM((2 * _CHUNK, _QS), f32),
            pltpu.VMEM((2 * _CHUNK, D_MSG), f32),
            pltpu.SemaphoreType.DMA((2,)),
            pltpu.SemaphoreType.DMA((2,)),
        ],
    )
    qs_g, d_g = gather(qs_table, d_table, src3, dstg3)

    idx5 = jnp.arange(512, dtype=jnp.int32)
    rep32 = (idx5[None, :] // 16 == jnp.arange(32, dtype=jnp.int32)[:, None]).astype(jnp.bfloat16)
    tile16 = (idx5[None, :] % 16 == jnp.arange(16, dtype=jnp.int32)[:, None]).astype(jnp.bfloat16)
    idx256 = jnp.arange(256, dtype=jnp.int32)
    shead = (idx256[:, None] // 32 == jnp.arange(8, dtype=jnp.int32)[None, :]).astype(jnp.bfloat16)
    rep8 = (idx256[None, :] // 32 == jnp.arange(8, dtype=jnp.int32)[:, None]).astype(jnp.bfloat16)
    rep8f = rep8.astype(f32)

    Eb = 1024
    evx = pl.pallas_call(
        _edge_body,
        grid=(epad // Eb,),
        in_specs=[
            pl.BlockSpec((Eb, _QS), lambda i: (i, 0)),
            pl.BlockSpec((Eb, D_MSG), lambda i: (i, 0)),
            pl.BlockSpec((Eb, N_BASIS), lambda i: (i, 0)),
            pl.BlockSpec((Eb, D_EDGE), lambda i: (i, 0)),
            pl.BlockSpec((N_BASIS, D_EDGE), lambda i: (0, 0)),
            pl.BlockSpec((1, D_EDGE), lambda i: (0, 0)),
            pl.BlockSpec((512, 512), lambda i: (0, 0)),
            pl.BlockSpec((32, 512), lambda i: (0, 0)),
            pl.BlockSpec((16, 512), lambda i: (0, 0)),
            pl.BlockSpec((256, 8), lambda i: (0, 0)),
            pl.BlockSpec((8, 256), lambda i: (0, 0)),
        ],
        out_specs=[pl.BlockSpec((Eb, 112), lambda i: (i, 0)),
                   pl.BlockSpec((Eb, 112), lambda i: (i, 0)),
                   pl.BlockSpec((Eb, 48), lambda i: (i, 0))],
        out_shape=[jax.ShapeDtypeStruct((epad, 112), f32),
                   jax.ShapeDtypeStruct((epad, 112), f32),
                   jax.ShapeDtypeStruct((epad, 48), f32)],
    )(qs_g, d_g, rbf_p, rsh_p, Wrbf, brbf.reshape(1, -1), Wkv,
      rep32, tile16, shead, rep8)
    ev_p0, ev_p1, ev_p2 = evx

    sjt = epad // _NW // _CHUNK

    def make_scatter(w):
        return pl.kernel(
            _scatter_body,
            out_type=jax.ShapeDtypeStruct((_NC, n, w), f32),
            mesh=plsc.VectorSubcoreMesh(core_axis_name="c",
                                        subcore_axis_name="s"),
            compiler_params=pltpu.CompilerParams(use_tc_tiling_on_sc=False),
            scratch_types=[
                pltpu.VMEM((sjt, _CHUNK), jnp.int32),
                pltpu.VMEM((_CHUNK, w), f32),
                pltpu.VMEM_SHARED((_NS * _STRIPE, w), f32),
                pltpu.SemaphoreType.DMA,
            ],
        )

    dsts3t = jnp.pad(dst_idx, (0, epad - E),
                     constant_values=n).reshape(_NW, sjt, _CHUNK)
    acc_p0 = make_scatter(112)(ev_p0, dsts3t, jnp.zeros((_STRIPE, 112), f32))
    acc_p1 = make_scatter(112)(ev_p1, dsts3t, jnp.zeros((_STRIPE, 112), f32))
    acc_p2 = make_scatter(48)(ev_p2, dsts3t, jnp.zeros((_STRIPE, 48), f32))

    out = pl.pallas_call(
        _node_body,
        grid=(n // Bn,),
        in_specs=[
            pl.BlockSpec((Bn, D_NODE), lambda i: (i, 0)),
            pl.BlockSpec((_NC, Bn, 112), lambda i: (0, i, 0)),
            pl.BlockSpec((_NC, Bn, 112), lambda i: (0, i, 0)),
            pl.BlockSpec((_NC, Bn, 48), lambda i: (0, i, 0)),
            pl.BlockSpec((D_NODE, D_NODE), lambda i: (0, 0)),
            pl.BlockSpec((1, D_NODE), lambda i: (0, 0)),
            pl.BlockSpec((1, D_NODE), lambda i: (0, 0)),
            pl.BlockSpec((D_NODE, D_NODE), lambda i: (0, 0)),
            pl.BlockSpec((1, D_NODE), lambda i: (0, 0)),
            pl.BlockSpec((D_NODE, D_NODE), lambda i: (0, 0)),
            pl.BlockSpec((1, D_NODE), lambda i: (0, 0)),
            pl.BlockSpec((8, 256), lambda i: (0, 0)),
        ],
        out_specs=pl.BlockSpec((Bn, D_NODE), lambda i: (i, 0)),
        out_shape=jax.ShapeDtypeStruct((n, D_NODE), f32),
    )(node, acc_p0, acc_p1, acc_p2, Wmsg,
      ln_w.reshape(1, -1), ln_b.reshape(1, -1),
      Wmlp1, bmlp1.reshape(1, -1), Wmlp2, bmlp2.reshape(1, -1), rep8f)
    return out
